# baseline (speedup 1.0000x reference)
import functools

import jax
import jax.numpy as jnp
from jax.experimental import pallas as pl
from jax.experimental.pallas import tpu as pltpu

_TM = 8


def _nn_pass_kernel(nt_total, lanes_ref, tile_ref, dist_ref, idx_ref,
                    rmin_ref, ridx_ref):
    n = lanes_ref.shape[2]
    mt = pl.program_id(1)

    @pl.when(mt == 0)
    def _init():
        rmin_ref[...] = jnp.full((_TM, n), jnp.inf, jnp.float32)
        ridx_ref[...] = jnp.zeros((_TM, n), jnp.int32)

    lanes = lanes_ref[0]
    tile = tile_ref[0]

    dx = tile[:, 0:1] - lanes[0:1, :]
    dy = tile[:, 1:2] - lanes[1:2, :]
    dz = tile[:, 2:3] - lanes[2:3, :]
    d = dx * dx + dy * dy + dz * dz

    iv = mt * _TM + jax.lax.broadcasted_iota(jnp.int32, (_TM, n), 0)
    cur = rmin_ref[...]
    take = d < cur
    rmin_ref[...] = jnp.where(take, d, cur)
    ridx_ref[...] = jnp.where(take, iv, ridx_ref[...])

    @pl.when(mt == nt_total - 1)
    def _fin():
        v = rmin_ref[...]
        ix = ridx_ref[...]
        h = _TM // 2
        while h >= 1:
            va, vb = v[:h], v[h:2 * h]
            ia, ib = ix[:h], ix[h:2 * h]
            bwin = (vb < va) | ((vb == va) & (ib < ia))
            v = jnp.where(bwin, vb, va)
            ix = jnp.where(bwin, ib, ia)
            h //= 2
        dist_ref[...] = v[None]
        idx_ref[...] = ix[None]


def _nn_pass(lanes_t, tile_pts):
    b, _, n = lanes_t.shape
    m = tile_pts.shape[1]
    nt = m // _TM
    dist, idx = pl.pallas_call(
        functools.partial(_nn_pass_kernel, nt),
        grid=(b, nt),
        in_specs=[
            pl.BlockSpec((1, 3, n), lambda bi, mt: (bi, 0, 0)),
            pl.BlockSpec((1, _TM, 3), lambda bi, mt: (bi, mt, 0)),
        ],
        out_specs=[
            pl.BlockSpec((1, 1, n), lambda bi, mt: (bi, 0, 0)),
            pl.BlockSpec((1, 1, n), lambda bi, mt: (bi, 0, 0)),
        ],
        out_shape=[
            jax.ShapeDtypeStruct((b, 1, n), jnp.float32),
            jax.ShapeDtypeStruct((b, 1, n), jnp.int32),
        ],
        scratch_shapes=[
            pltpu.VMEM((_TM, n), jnp.float32),
            pltpu.VMEM((_TM, n), jnp.int32),
        ],
    )(lanes_t, tile_pts)
    return dist[:, 0], idx[:, 0]


def kernel(input1, input2):
    x1t = jnp.transpose(input1, (0, 2, 1))
    x2t = jnp.transpose(input2, (0, 2, 1))
    dist1, idx1 = _nn_pass(x1t, input2)
    dist2, idx2 = _nn_pass(x2t, input1)
    return (dist1, idx1, dist2, idx2)


# speedup vs baseline: 5.7686x; 5.7686x over previous
import functools

import jax
import jax.numpy as jnp
from jax import lax
from jax.experimental import pallas as pl
from jax.experimental.pallas import tpu as pltpu

_MC = 512
_U = 4
_L = 128


def _cd_kernel(n_chunks, mc, q_ref, db_ref, dist_ref, idx_ref,
               rmin_ref, ridx_ref):
    c = pl.program_id(2)
    s = q_ref.shape[3]

    @pl.when(c == 0)
    def _init():
        rmin_ref[...] = jnp.full((s, _L), jnp.inf, jnp.float32)
        ridx_ref[...] = jnp.zeros((s, _L), jnp.int32)

    qx = q_ref[0, 0, 0]
    qy = q_ref[0, 0, 1]
    qz = q_ref[0, 0, 2]
    base = c * mc

    def body(j, carry):
        rmin, ridx = carry
        for k in range(_U):
            i = j * _U + k
            px = db_ref[0, 0, i, 0]
            py = db_ref[0, 0, i, 1]
            pz = db_ref[0, 0, i, 2]
            dx = qx - px
            dy = qy - py
            dz = qz - pz
            d = dx * dx + dy * dy + dz * dz
            take = d < rmin
            rmin = jnp.where(take, d, rmin)
            ridx = jnp.where(take, base + i, ridx)
        return rmin, ridx

    rmin, ridx = lax.fori_loop(
        0, mc // _U, body, (rmin_ref[...], ridx_ref[...]))
    rmin_ref[...] = rmin
    ridx_ref[...] = ridx

    @pl.when(c == n_chunks - 1)
    def _fin():
        dist_ref[0, 0] = rmin_ref[...]
        idx_ref[0, 0] = ridx_ref[...]


def _chamfer_both(q, db):
    _, b, _, s, _ = q.shape
    m = db.shape[2]
    mc = min(_MC, m)
    n_chunks = m // mc
    dist, idx = pl.pallas_call(
        functools.partial(_cd_kernel, n_chunks, mc),
        grid=(2, b, n_chunks),
        in_specs=[
            pl.BlockSpec((1, 1, 3, s, _L), lambda p, bi, c: (p, bi, 0, 0, 0)),
            pl.BlockSpec((1, 1, mc, 3), lambda p, bi, c: (p, bi, c, 0),
                         memory_space=pltpu.SMEM),
        ],
        out_specs=[
            pl.BlockSpec((1, 1, s, _L), lambda p, bi, c: (p, bi, 0, 0)),
            pl.BlockSpec((1, 1, s, _L), lambda p, bi, c: (p, bi, 0, 0)),
        ],
        out_shape=[
            jax.ShapeDtypeStruct((2, b, s, _L), jnp.float32),
            jax.ShapeDtypeStruct((2, b, s, _L), jnp.int32),
        ],
        scratch_shapes=[
            pltpu.VMEM((s, _L), jnp.float32),
            pltpu.VMEM((s, _L), jnp.int32),
        ],
    )(q, db)
    return dist, idx


def kernel(input1, input2):
    b, n, _ = input1.shape
    s = n // _L
    q1 = jnp.transpose(input1, (0, 2, 1)).reshape(b, 3, s, _L)
    q2 = jnp.transpose(input2, (0, 2, 1)).reshape(b, 3, s, _L)
    q = jnp.stack([q1, q2])
    db = jnp.stack([input2, input1])
    dist, idx = _chamfer_both(q, db)
    dist1 = dist[0].reshape(b, n)
    idx1 = idx[0].reshape(b, n)
    dist2 = dist[1].reshape(b, n)
    idx2 = idx[1].reshape(b, n)
    return (dist1, idx1, dist2, idx2)


# speedup vs baseline: 7.3305x; 1.2708x over previous
import functools

import jax
import jax.numpy as jnp
from jax import lax
from jax.experimental import pallas as pl
from jax.experimental.pallas import tpu as pltpu

_MC = 512
_U = 4
_L = 128


def _cd_kernel(n_chunks, mc, q_ref, dbe_ref, iv_ref, dist_ref, idx_ref,
               rmin_ref, ridx_ref):
    c = pl.program_id(2)
    s = q_ref.shape[3]

    @pl.when(c == 0)
    def _init():
        rmin_ref[...] = jnp.full((s, _L), jnp.inf, jnp.float32)
        ridx_ref[...] = jnp.zeros((s, _L), jnp.int32)

    qx = q_ref[0, 0, 0]
    qy = q_ref[0, 0, 1]
    qz = q_ref[0, 0, 2]

    def body(j, carry):
        rmin, ridx = carry
        for k in range(_U):
            i = j * _U + k
            px = dbe_ref[0, 0, 0, pl.ds(i, 1), :]
            py = dbe_ref[0, 0, 1, pl.ds(i, 1), :]
            pz = dbe_ref[0, 0, 2, pl.ds(i, 1), :]
            iv = iv_ref[pl.ds(i, 1), :]
            dx = qx - px
            dy = qy - py
            dz = qz - pz
            d = dx * dx + dy * dy + dz * dz
            take = d < rmin
            rmin = jnp.where(take, d, rmin)
            ridx = jnp.where(take, iv, ridx)
        return rmin, ridx

    rmin, ridx = lax.fori_loop(
        0, mc // _U, body, (rmin_ref[...], ridx_ref[...]))
    rmin_ref[...] = rmin
    ridx_ref[...] = ridx

    @pl.when(c == n_chunks - 1)
    def _fin():
        dist_ref[0, 0] = rmin_ref[...]
        idx_ref[0, 0] = ridx_ref[...]


def _chamfer_both(q, dbe, iv):
    _, b, _, s, _ = q.shape
    m = dbe.shape[3]
    mc = min(_MC, m)
    n_chunks = m // mc
    dist, idx = pl.pallas_call(
        functools.partial(_cd_kernel, n_chunks, mc),
        grid=(2, b, n_chunks),
        in_specs=[
            pl.BlockSpec((1, 1, 3, s, _L), lambda p, bi, c: (p, bi, 0, 0, 0)),
            pl.BlockSpec((1, 1, 3, mc, _L),
                         lambda p, bi, c: (p, bi, 0, c, 0)),
            pl.BlockSpec((mc, _L), lambda p, bi, c: (c, 0)),
        ],
        out_specs=[
            pl.BlockSpec((1, 1, s, _L), lambda p, bi, c: (p, bi, 0, 0)),
            pl.BlockSpec((1, 1, s, _L), lambda p, bi, c: (p, bi, 0, 0)),
        ],
        out_shape=[
            jax.ShapeDtypeStruct((2, b, s, _L), jnp.float32),
            jax.ShapeDtypeStruct((2, b, s, _L), jnp.int32),
        ],
        scratch_shapes=[
            pltpu.VMEM((s, _L), jnp.float32),
            pltpu.VMEM((s, _L), jnp.int32),
        ],
    )(q, dbe, iv)
    return dist, idx


def kernel(input1, input2):
    b, n, _ = input1.shape
    s = n // _L
    x1t = jnp.transpose(input1, (0, 2, 1))
    x2t = jnp.transpose(input2, (0, 2, 1))
    q = jnp.stack([x1t.reshape(b, 3, s, _L), x2t.reshape(b, 3, s, _L)])
    db = jnp.stack([x2t, x1t])
    dbe = jnp.broadcast_to(db[..., None], db.shape + (_L,))
    iv = jnp.broadcast_to(
        jnp.arange(n, dtype=jnp.int32)[:, None], (n, _L))
    dist, idx = _chamfer_both(q, dbe, iv)
    dist1 = dist[0].reshape(b, n)
    idx1 = idx[0].reshape(b, n)
    dist2 = dist[1].reshape(b, n)
    idx2 = idx[1].reshape(b, n)
    return (dist1, idx1, dist2, idx2)


# speedup vs baseline: 7.6618x; 1.0452x over previous
import functools

import jax
import jax.numpy as jnp
from jax import lax
from jax.experimental import pallas as pl
from jax.experimental.pallas import tpu as pltpu

_MC = 512
_U = 8
_L = 128


def _cd_kernel(n_chunks, mc, q_ref, dbe_ref, iv_ref, dist_ref, idx_ref,
               rmin_ref, ridx_ref):
    c = pl.program_id(2)
    s = q_ref.shape[3]

    @pl.when(c == 0)
    def _init():
        rmin_ref[...] = jnp.full((s, _L), jnp.inf, jnp.float32)
        ridx_ref[...] = jnp.zeros((s, _L), jnp.int32)

    qx = q_ref[0, 0, 0]
    qy = q_ref[0, 0, 1]
    qz = q_ref[0, 0, 2]

    def body(j, carry):
        rmin, ridx = carry
        base = j * _U
        pxb = dbe_ref[0, 0, 0, pl.ds(base, _U), :]
        pyb = dbe_ref[0, 0, 1, pl.ds(base, _U), :]
        pzb = dbe_ref[0, 0, 2, pl.ds(base, _U), :]
        ivb = iv_ref[pl.ds(base, _U), :]
        for k in range(_U):
            dx = qx - pxb[k:k + 1, :]
            dy = qy - pyb[k:k + 1, :]
            dz = qz - pzb[k:k + 1, :]
            d = dx * dx + dy * dy + dz * dz
            take = d < rmin
            rmin = jnp.where(take, d, rmin)
            ridx = jnp.where(take, ivb[k:k + 1, :], ridx)
        return rmin, ridx

    rmin, ridx = lax.fori_loop(
        0, mc // _U, body, (rmin_ref[...], ridx_ref[...]))
    rmin_ref[...] = rmin
    ridx_ref[...] = ridx

    @pl.when(c == n_chunks - 1)
    def _fin():
        dist_ref[0, 0] = rmin_ref[...]
        idx_ref[0, 0] = ridx_ref[...]


def _chamfer_both(q, dbe, iv):
    _, b, _, s, _ = q.shape
    m = dbe.shape[3]
    mc = min(_MC, m)
    n_chunks = m // mc
    dist, idx = pl.pallas_call(
        functools.partial(_cd_kernel, n_chunks, mc),
        grid=(2, b, n_chunks),
        in_specs=[
            pl.BlockSpec((1, 1, 3, s, _L), lambda p, bi, c: (p, bi, 0, 0, 0)),
            pl.BlockSpec((1, 1, 3, mc, _L),
                         lambda p, bi, c: (p, bi, 0, c, 0)),
            pl.BlockSpec((mc, _L), lambda p, bi, c: (c, 0)),
        ],
        out_specs=[
            pl.BlockSpec((1, 1, s, _L), lambda p, bi, c: (p, bi, 0, 0)),
            pl.BlockSpec((1, 1, s, _L), lambda p, bi, c: (p, bi, 0, 0)),
        ],
        out_shape=[
            jax.ShapeDtypeStruct((2, b, s, _L), jnp.float32),
            jax.ShapeDtypeStruct((2, b, s, _L), jnp.int32),
        ],
        scratch_shapes=[
            pltpu.VMEM((s, _L), jnp.float32),
            pltpu.VMEM((s, _L), jnp.int32),
        ],
    )(q, dbe, iv)
    return dist, idx


def kernel(input1, input2):
    b, n, _ = input1.shape
    s = n // _L
    x1t = jnp.transpose(input1, (0, 2, 1))
    x2t = jnp.transpose(input2, (0, 2, 1))
    q = jnp.stack([x1t.reshape(b, 3, s, _L), x2t.reshape(b, 3, s, _L)])
    db = jnp.stack([x2t, x1t])
    dbe = jnp.broadcast_to(db[..., None], db.shape + (_L,))
    iv = jnp.broadcast_to(
        jnp.arange(n, dtype=jnp.int32)[:, None], (n, _L))
    dist, idx = _chamfer_both(q, dbe, iv)
    dist1 = dist[0].reshape(b, n)
    idx1 = idx[0].reshape(b, n)
    dist2 = dist[1].reshape(b, n)
    idx2 = idx[1].reshape(b, n)
    return (dist1, idx1, dist2, idx2)


# speedup vs baseline: 7.9105x; 1.0325x over previous
import functools

import jax
import jax.numpy as jnp
from jax import lax
from jax.experimental import pallas as pl
from jax.experimental.pallas import tpu as pltpu

_MC = 512
_U = 16
_L = 128


def _cd_kernel(n_chunks, mc, q_ref, dbe_ref, iv_ref, dist_ref, idx_ref,
               rmin_ref, ridx_ref):
    c = pl.program_id(2)
    s = q_ref.shape[3]

    @pl.when(c == 0)
    def _init():
        rmin_ref[...] = jnp.full((s, _L), jnp.inf, jnp.float32)
        ridx_ref[...] = jnp.zeros((s, _L), jnp.int32)

    qx = q_ref[0, 0, 0]
    qy = q_ref[0, 0, 1]
    qz = q_ref[0, 0, 2]

    def body(j, carry):
        rmin, ridx = carry
        base = j * _U
        pxb = dbe_ref[0, 0, 0, pl.ds(base, _U), :]
        pyb = dbe_ref[0, 0, 1, pl.ds(base, _U), :]
        pzb = dbe_ref[0, 0, 2, pl.ds(base, _U), :]
        ivb = iv_ref[pl.ds(base, _U), :]
        for k in range(_U):
            dx = qx - pxb[k:k + 1, :]
            dy = qy - pyb[k:k + 1, :]
            dz = qz - pzb[k:k + 1, :]
            d = dx * dx + dy * dy + dz * dz
            take = d < rmin
            rmin = jnp.where(take, d, rmin)
            ridx = jnp.where(take, ivb[k:k + 1, :], ridx)
        return rmin, ridx

    rmin, ridx = lax.fori_loop(
        0, mc // _U, body, (rmin_ref[...], ridx_ref[...]))
    rmin_ref[...] = rmin
    ridx_ref[...] = ridx

    @pl.when(c == n_chunks - 1)
    def _fin():
        dist_ref[0, 0] = rmin_ref[...]
        idx_ref[0, 0] = ridx_ref[...]


def _chamfer_both(q, dbe, iv):
    _, b, _, s, _ = q.shape
    m = dbe.shape[3]
    mc = min(_MC, m)
    n_chunks = m // mc
    dist, idx = pl.pallas_call(
        functools.partial(_cd_kernel, n_chunks, mc),
        grid=(2, b, n_chunks),
        in_specs=[
            pl.BlockSpec((1, 1, 3, s, _L), lambda p, bi, c: (p, bi, 0, 0, 0)),
            pl.BlockSpec((1, 1, 3, mc, _L),
                         lambda p, bi, c: (p, bi, 0, c, 0)),
            pl.BlockSpec((mc, _L), lambda p, bi, c: (c, 0)),
        ],
        out_specs=[
            pl.BlockSpec((1, 1, s, _L), lambda p, bi, c: (p, bi, 0, 0)),
            pl.BlockSpec((1, 1, s, _L), lambda p, bi, c: (p, bi, 0, 0)),
        ],
        out_shape=[
            jax.ShapeDtypeStruct((2, b, s, _L), jnp.float32),
            jax.ShapeDtypeStruct((2, b, s, _L), jnp.int32),
        ],
        scratch_shapes=[
            pltpu.VMEM((s, _L), jnp.float32),
            pltpu.VMEM((s, _L), jnp.int32),
        ],
    )(q, dbe, iv)
    return dist, idx


def kernel(input1, input2):
    b, n, _ = input1.shape
    s = n // _L
    x1t = jnp.transpose(input1, (0, 2, 1))
    x2t = jnp.transpose(input2, (0, 2, 1))
    q = jnp.stack([x1t.reshape(b, 3, s, _L), x2t.reshape(b, 3, s, _L)])
    db = jnp.stack([x2t, x1t])
    dbe = jnp.broadcast_to(db[..., None], db.shape + (_L,))
    iv = jnp.broadcast_to(
        jnp.arange(n, dtype=jnp.int32)[:, None], (n, _L))
    dist, idx = _chamfer_both(q, dbe, iv)
    dist1 = dist[0].reshape(b, n)
    idx1 = idx[0].reshape(b, n)
    dist2 = dist[1].reshape(b, n)
    idx2 = idx[1].reshape(b, n)
    return (dist1, idx1, dist2, idx2)


# speedup vs baseline: 8.0196x; 1.0138x over previous
import functools

import jax
import jax.numpy as jnp
from jax import lax
from jax.experimental import pallas as pl
from jax.experimental.pallas import tpu as pltpu

_MC = 4096
_U = 16
_L = 128


def _cd_kernel(n_chunks, mc, q_ref, dbe_ref, iv_ref, dist_ref, idx_ref,
               rmin_ref, ridx_ref):
    c = pl.program_id(2)
    s = q_ref.shape[3]

    @pl.when(c == 0)
    def _init():
        rmin_ref[...] = jnp.full((s, _L), jnp.inf, jnp.float32)
        ridx_ref[...] = jnp.zeros((s, _L), jnp.int32)

    qx = q_ref[0, 0, 0]
    qy = q_ref[0, 0, 1]
    qz = q_ref[0, 0, 2]

    def body(j, carry):
        rmin, ridx = carry
        base = j * _U
        pxb = dbe_ref[0, 0, 0, pl.ds(base, _U), :]
        pyb = dbe_ref[0, 0, 1, pl.ds(base, _U), :]
        pzb = dbe_ref[0, 0, 2, pl.ds(base, _U), :]
        ivb = iv_ref[pl.ds(base, _U), :]
        for k in range(_U):
            dx = qx - pxb[k:k + 1, :]
            dy = qy - pyb[k:k + 1, :]
            dz = qz - pzb[k:k + 1, :]
            d = dx * dx + dy * dy + dz * dz
            take = d < rmin
            rmin = jnp.where(take, d, rmin)
            ridx = jnp.where(take, ivb[k:k + 1, :], ridx)
        return rmin, ridx

    rmin, ridx = lax.fori_loop(
        0, mc // _U, body, (rmin_ref[...], ridx_ref[...]))
    rmin_ref[...] = rmin
    ridx_ref[...] = ridx

    @pl.when(c == n_chunks - 1)
    def _fin():
        dist_ref[0, 0] = rmin_ref[...]
        idx_ref[0, 0] = ridx_ref[...]


def _chamfer_both(q, dbe, iv):
    _, b, _, s, _ = q.shape
    m = dbe.shape[3]
    mc = min(_MC, m)
    n_chunks = m // mc
    dist, idx = pl.pallas_call(
        functools.partial(_cd_kernel, n_chunks, mc),
        grid=(2, b, n_chunks),
        in_specs=[
            pl.BlockSpec((1, 1, 3, s, _L), lambda p, bi, c: (p, bi, 0, 0, 0)),
            pl.BlockSpec((1, 1, 3, mc, _L),
                         lambda p, bi, c: (p, bi, 0, c, 0)),
            pl.BlockSpec((mc, _L), lambda p, bi, c: (c, 0)),
        ],
        out_specs=[
            pl.BlockSpec((1, 1, s, _L), lambda p, bi, c: (p, bi, 0, 0)),
            pl.BlockSpec((1, 1, s, _L), lambda p, bi, c: (p, bi, 0, 0)),
        ],
        out_shape=[
            jax.ShapeDtypeStruct((2, b, s, _L), jnp.float32),
            jax.ShapeDtypeStruct((2, b, s, _L), jnp.int32),
        ],
        scratch_shapes=[
            pltpu.VMEM((s, _L), jnp.float32),
            pltpu.VMEM((s, _L), jnp.int32),
        ],
    )(q, dbe, iv)
    return dist, idx


def kernel(input1, input2):
    b, n, _ = input1.shape
    s = n // _L
    x1t = jnp.transpose(input1, (0, 2, 1))
    x2t = jnp.transpose(input2, (0, 2, 1))
    q = jnp.stack([x1t.reshape(b, 3, s, _L), x2t.reshape(b, 3, s, _L)])
    db = jnp.stack([x2t, x1t])
    dbe = jnp.broadcast_to(db[..., None], db.shape + (_L,))
    iv = jnp.broadcast_to(
        jnp.arange(n, dtype=jnp.int32)[:, None], (n, _L))
    dist, idx = _chamfer_both(q, dbe, iv)
    dist1 = dist[0].reshape(b, n)
    idx1 = idx[0].reshape(b, n)
    dist2 = dist[1].reshape(b, n)
    idx2 = idx[1].reshape(b, n)
    return (dist1, idx1, dist2, idx2)
